# Initial kernel scaffold; baseline (speedup 1.0000x reference)
import jax
import jax.numpy as jnp
from jax.experimental import pallas as pl


def kernel(x, edge_index, W1_l, b1, W1_r, W2_l, b2, W2_r):
    raise NotImplementedError("write your pallas kernel here")


# speedup vs baseline: 13.1480x; 13.1480x over previous
import functools

import jax
import jax.numpy as jnp
from jax import lax
from jax.experimental import pallas as pl
from jax.experimental.pallas import tpu as pltpu
from jax.experimental.pallas import tpu_sc as plsc

NC = 2
NS = 16
NW = NC * NS
CHUNK = 128
CW = 80
NBUF = 4
AHEAD = 2
RB = 1280


def _sc_segment_sum(dim, with_count):
  def body(h_hbm, src_hbm, dst_hbm, *rest):
    if with_count:
      agg_out, cnt_out = rest[0], rest[1]
      rest = rest[2:]
    else:
      agg_out = rest[0]
      cnt_out = None
      rest = rest[1:]
    idx_src, idx_dst, rows_v, ones_v, zcnt_v = rest[:5]
    gsem = rest[5:5 + NBUF]
    ssem = rest[5 + NBUF:5 + 2 * NBUF]
    acc_sh = rest[5 + 2 * NBUF]
    cnt_sh = rest[6 + 2 * NBUF] if with_count else None

    np_rows = h_hbm.shape[0]
    rows_per_tile = np_rows // NS

    c = lax.axis_index("c")
    s = lax.axis_index("s")
    wid = s * NC + c

    pltpu.sync_copy(src_hbm.at[pl.ds(wid * CW, CW), :], idx_src)
    pltpu.sync_copy(dst_hbm.at[pl.ds(wid * CW, CW), :], idx_dst)

    zf = jnp.zeros((16,), jnp.float32)
    of = jnp.ones((16,), jnp.float32)

    def init_row(r, carry):
      for j in range(dim // 16):
        rows_v[0, r, pl.ds(j * 16, 16)] = zf
      ones_v[r, pl.ds(0, 16)] = of
      zcnt_v[r, pl.ds(0, 16)] = zf
      return carry

    lax.fori_loop(0, CHUNK, init_row, 0)

    tbase = s * rows_per_tile
    for k in range(rows_per_tile // CHUNK):
      pltpu.sync_copy(rows_v.at[0],
                      acc_sh.at[pl.ds(tbase + k * CHUNK, CHUNK), :])
      if with_count:
        pltpu.sync_copy(zcnt_v,
                        cnt_sh.at[pl.ds(tbase + k * CHUNK, CHUNK), :])
    plsc.subcore_barrier()

    def gather_start(i, b):
      pltpu.async_copy(h_hbm.at[idx_src.at[i]], rows_v.at[b], gsem[b])

    def gather_wait(i, b):
      pltpu.make_async_copy(h_hbm.at[idx_src.at[i]], rows_v.at[b],
                            gsem[b]).wait()

    def scatter_start(i, b):
      pltpu.async_copy(rows_v.at[b], acc_sh.at[idx_dst.at[i]], ssem[b],
                       add=True)

    def scatter_wait(i, b):
      pltpu.make_async_copy(rows_v.at[b], acc_sh.at[idx_dst.at[i]],
                            ssem[b]).wait()

    def step(i, b, first, last):
      j = i + AHEAD
      bj = (b + AHEAD) % NBUF
      if not first and not last:
        scatter_wait(j - NBUF, bj)
      if not last:
        gather_start(j, bj)
      gather_wait(i, b)
      scatter_start(i, b)
      if with_count:
        pltpu.sync_copy(ones_v, cnt_sh.at[idx_dst.at[i]], add=True)

    for i in range(AHEAD):
      gather_start(i, i % NBUF)

    for b in range(NBUF):
      step(b, b, first=(b < NBUF - AHEAD), last=False)

    def group(g, carry):
      for b in range(NBUF):
        step(g * NBUF + b, b, first=False, last=False)
      return carry

    lax.fori_loop(1, CW // NBUF - 1, group, 0)

    g = CW // NBUF - 1
    for b in range(NBUF):
      i = g * NBUF + b
      step(i, b, first=False, last=(i + AHEAD >= CW))

    for i in range(CW - NBUF, CW):
      scatter_wait(i, i % NBUF)

    plsc.subcore_barrier()

    pltpu.sync_copy(acc_sh.at[pl.ds(tbase, rows_per_tile), :],
                    agg_out.at[c, pl.ds(tbase, rows_per_tile), :])
    if with_count:
      pltpu.sync_copy(cnt_sh.at[pl.ds(tbase, rows_per_tile), :],
                      cnt_out.at[c, pl.ds(tbase, rows_per_tile), :])

  def call(h, src2d, dst2d):
    np_rows = h.shape[0]
    out_type = [jax.ShapeDtypeStruct((NC, np_rows, dim), jnp.float32)]
    if with_count:
      out_type.append(jax.ShapeDtypeStruct((NC, np_rows, 16), jnp.float32))
    scratch = [
        pltpu.VMEM((CW, CHUNK), jnp.int32),
        pltpu.VMEM((CW, CHUNK), jnp.int32),
        pltpu.VMEM((NBUF, CHUNK, dim), jnp.float32),
        pltpu.VMEM((CHUNK, 16), jnp.float32),
        pltpu.VMEM((CHUNK, 16), jnp.float32),
    ]
    scratch += [pltpu.SemaphoreType.DMA] * (2 * NBUF)
    scratch.append(pltpu.VMEM_SHARED((np_rows, dim), jnp.float32))
    if with_count:
      scratch.append(pltpu.VMEM_SHARED((np_rows, 16), jnp.float32))
    fn = pl.kernel(
        body,
        out_type=tuple(out_type),
        mesh=plsc.VectorSubcoreMesh(core_axis_name="c", subcore_axis_name="s",
                                    num_cores=NC, num_subcores=NS),
        scratch_types=tuple(scratch),
        compiler_params=pltpu.CompilerParams(use_tc_tiling_on_sc=False),
    )
    return fn(h, src2d, dst2d)

  return call


def _tc_project(x, wl, wr, b):
  rows, d = x.shape
  h = wl.shape[1]
  hh = h // 2

  def body(x_ref, wl_ref, wr_ref, b_ref, ha_ref, hb_ref, r_ref):
    xb = x_ref[...]
    hv = jnp.dot(xb, wl_ref[...], preferred_element_type=jnp.float32)
    ha_ref[...] = hv[:, :hh]
    hb_ref[...] = hv[:, hh:]
    r_ref[...] = (jnp.dot(xb, wr_ref[...], preferred_element_type=jnp.float32)
                  + b_ref[...])

  return pl.pallas_call(
      body,
      grid=(rows // RB,),
      in_specs=[
          pl.BlockSpec((RB, d), lambda i: (i, 0)),
          pl.BlockSpec((d, h), lambda i: (0, 0)),
          pl.BlockSpec((d, h), lambda i: (0, 0)),
          pl.BlockSpec((1, h), lambda i: (0, 0)),
      ],
      out_specs=[
          pl.BlockSpec((RB, hh), lambda i: (i, 0)),
          pl.BlockSpec((RB, hh), lambda i: (i, 0)),
          pl.BlockSpec((RB, h), lambda i: (i, 0)),
      ],
      out_shape=[
          jax.ShapeDtypeStruct((rows, hh), jnp.float32),
          jax.ShapeDtypeStruct((rows, hh), jnp.float32),
          jax.ShapeDtypeStruct((rows, h), jnp.float32),
      ],
  )(x, wl, wr, b)


def _tc_mid(agga, aggb, cnt, r1, wl, wr, b):
  rows, d = r1.shape
  dh = d // 2
  h = wl.shape[1]

  def body(agga_ref, aggb_ref, cnt_ref, r1_ref, wl_ref, wr_ref, b_ref,
           h_ref, r_ref):
    cb = cnt_ref[0] + cnt_ref[1]
    inv = 1.0 / jnp.maximum(cb[:, :1], 1.0)
    mean = jnp.concatenate(
        [agga_ref[0] + agga_ref[1], aggb_ref[0] + aggb_ref[1]], axis=1) * inv
    z = jnp.maximum(mean + r1_ref[...], 0.0)
    h_ref[...] = jnp.dot(z, wl_ref[...], preferred_element_type=jnp.float32)
    r_ref[...] = (jnp.dot(z, wr_ref[...], preferred_element_type=jnp.float32)
                  + b_ref[...])

  return pl.pallas_call(
      body,
      grid=(rows // RB,),
      in_specs=[
          pl.BlockSpec((NC, RB, dh), lambda i: (0, i, 0)),
          pl.BlockSpec((NC, RB, dh), lambda i: (0, i, 0)),
          pl.BlockSpec((NC, RB, 16), lambda i: (0, i, 0)),
          pl.BlockSpec((RB, d), lambda i: (i, 0)),
          pl.BlockSpec((d, h), lambda i: (0, 0)),
          pl.BlockSpec((d, h), lambda i: (0, 0)),
          pl.BlockSpec((1, h), lambda i: (0, 0)),
      ],
      out_specs=[
          pl.BlockSpec((RB, h), lambda i: (i, 0)),
          pl.BlockSpec((RB, h), lambda i: (i, 0)),
      ],
      out_shape=[
          jax.ShapeDtypeStruct((rows, h), jnp.float32),
          jax.ShapeDtypeStruct((rows, h), jnp.float32),
      ],
  )(agga, aggb, cnt, r1, wl, wr, b)


def _tc_final(agg, cnt, r2):
  rows, h = r2.shape

  def body(agg_ref, cnt_ref, r2_ref, o_ref):
    cb = cnt_ref[0] + cnt_ref[1]
    inv = 1.0 / jnp.maximum(cb[:, :1], 1.0)
    o_ref[...] = (agg_ref[0] + agg_ref[1]) * inv + r2_ref[...]

  return pl.pallas_call(
      body,
      grid=(rows // RB,),
      in_specs=[
          pl.BlockSpec((NC, RB, h), lambda i: (0, i, 0)),
          pl.BlockSpec((NC, RB, 16), lambda i: (0, i, 0)),
          pl.BlockSpec((RB, h), lambda i: (i, 0)),
      ],
      out_specs=pl.BlockSpec((RB, h), lambda i: (i, 0)),
      out_shape=jax.ShapeDtypeStruct((rows, h), jnp.float32),
  )(agg, cnt, r2)


@jax.jit
def kernel(x, edge_index, W1_l, b1, W1_r, W2_l, b2, W2_r):
  n, d = x.shape
  e = edge_index.shape[1]

  np_rows = ((n + 240 + NS * CHUNK - 1) // (NS * CHUNK)) * (NS * CHUNK)
  ep = ((e + NW * CW * CHUNK - 1) // (NW * CW * CHUNK)) * (NW * CW * CHUNK)
  pad_e = ep - e
  n_dummy = np_rows - n

  src = edge_index[0]
  dst = edge_index[1]
  pad_ids = jnp.arange(pad_e, dtype=jnp.int32)
  src_p = jnp.concatenate([src, pad_ids % n])
  dst_p = jnp.concatenate([dst, n + pad_ids % n_dummy])
  src2d = src_p.reshape(NW * CW * (ep // (NW * CW * CHUNK)), CHUNK)
  dst2d = dst_p.reshape(src2d.shape)

  x_p = jnp.pad(x, ((0, np_rows - n), (0, 0)))
  b1r = b1.reshape(1, -1)
  b2r = b2.reshape(1, -1)

  h1a, h1b, r1 = _tc_project(x_p, W1_l.T, W1_r.T, b1r)
  seg = _sc_segment_sum(h1a.shape[1], True)
  seg_nc = _sc_segment_sum(h1a.shape[1], False)
  agg1a, cnt = seg(h1a, src2d, dst2d)
  (agg1b,) = seg_nc(h1b, src2d, dst2d)
  h2, r2 = _tc_mid(agg1a, agg1b, cnt, r1, W2_l.T, W2_r.T, b2r)
  (agg2,) = seg_nc(h2, src2d, dst2d)
  out = _tc_final(agg2, cnt, r2)
  return out[:n]


# speedup vs baseline: 13.2105x; 1.0047x over previous
import functools

import jax
import jax.numpy as jnp
import numpy as np
from jax import lax
from jax.experimental import pallas as pl
from jax.experimental.pallas import tpu as pltpu
from jax.experimental.pallas import tpu_sc as plsc

NC = 2
NS = 16
NW = NC * NS
CHUNK = 128
CW = 80
NBUF = 4
AHEAD = 2
RB = 1280


def _sc_segment_sum(dim, with_count):
  def body(h_hbm, src_hbm, dst_hbm, *rest):
    if with_count:
      agg_out, cnt_out = rest[0], rest[1]
      rest = rest[2:]
    else:
      agg_out = rest[0]
      cnt_out = None
      rest = rest[1:]
    idx_src, idx_dst, rows_v, ones_v, zcnt_v = rest[:5]
    gsem = rest[5:5 + NBUF]
    ssem = rest[5 + NBUF:5 + 2 * NBUF]
    acc_sh = rest[5 + 2 * NBUF]
    cnt_sh = rest[6 + 2 * NBUF] if with_count else None

    np_rows = h_hbm.shape[0]
    rows_per_tile = np_rows // NS

    c = lax.axis_index("c")
    s = lax.axis_index("s")
    wid = s * NC + c

    pltpu.sync_copy(src_hbm.at[pl.ds(wid * CW, CW), :], idx_src)
    pltpu.sync_copy(dst_hbm.at[pl.ds(wid * CW, CW), :], idx_dst)

    zf = jnp.zeros((16,), jnp.float32)
    of = jnp.ones((16,), jnp.float32)

    def init_row(r, carry):
      for j in range(dim // 16):
        rows_v[0, r, pl.ds(j * 16, 16)] = zf
      ones_v[r, pl.ds(0, 16)] = of
      zcnt_v[r, pl.ds(0, 16)] = zf
      return carry

    lax.fori_loop(0, CHUNK, init_row, 0)

    tbase = s * rows_per_tile
    for k in range(rows_per_tile // CHUNK):
      pltpu.sync_copy(rows_v.at[0],
                      acc_sh.at[pl.ds(tbase + k * CHUNK, CHUNK), :])
      if with_count:
        pltpu.sync_copy(zcnt_v,
                        cnt_sh.at[pl.ds(tbase + k * CHUNK, CHUNK), :])
    plsc.subcore_barrier()

    def gather_start(i, b):
      pltpu.async_copy(h_hbm.at[idx_src.at[i]], rows_v.at[b], gsem[b])

    def gather_wait(i, b):
      pltpu.make_async_copy(h_hbm.at[idx_src.at[i]], rows_v.at[b],
                            gsem[b]).wait()

    def scatter_start(i, b):
      pltpu.async_copy(rows_v.at[b], acc_sh.at[idx_dst.at[i]], ssem[b],
                       add=True)

    def scatter_wait(i, b):
      pltpu.make_async_copy(rows_v.at[b], acc_sh.at[idx_dst.at[i]],
                            ssem[b]).wait()

    def step(i, b, first, last):
      j = i + AHEAD
      bj = (b + AHEAD) % NBUF
      if not first and not last:
        scatter_wait(j - NBUF, bj)
      if not last:
        gather_start(j, bj)
      gather_wait(i, b)
      scatter_start(i, b)
      if with_count:
        pltpu.sync_copy(ones_v, cnt_sh.at[idx_dst.at[i]], add=True)

    for i in range(AHEAD):
      gather_start(i, i % NBUF)

    for b in range(NBUF):
      step(b, b, first=(b < NBUF - AHEAD), last=False)

    def group(g, carry):
      for b in range(NBUF):
        step(g * NBUF + b, b, first=False, last=False)
      return carry

    lax.fori_loop(1, CW // NBUF - 1, group, 0)

    g = CW // NBUF - 1
    for b in range(NBUF):
      i = g * NBUF + b
      step(i, b, first=False, last=(i + AHEAD >= CW))

    for i in range(CW - NBUF, CW):
      scatter_wait(i, i % NBUF)

    plsc.subcore_barrier()

    pltpu.sync_copy(acc_sh.at[pl.ds(tbase, rows_per_tile), :],
                    agg_out.at[c, pl.ds(tbase, rows_per_tile), :])
    if with_count:
      pltpu.sync_copy(cnt_sh.at[pl.ds(tbase, rows_per_tile), :],
                      cnt_out.at[c, pl.ds(tbase, rows_per_tile), :])

  def call(h, src2d, dst2d):
    np_rows = h.shape[0]
    out_type = [jax.ShapeDtypeStruct((NC, np_rows, dim), jnp.float32)]
    if with_count:
      out_type.append(jax.ShapeDtypeStruct((NC, np_rows, 16), jnp.float32))
    scratch = [
        pltpu.VMEM((CW, CHUNK), jnp.int32),
        pltpu.VMEM((CW, CHUNK), jnp.int32),
        pltpu.VMEM((NBUF, CHUNK, dim), jnp.float32),
        pltpu.VMEM((CHUNK, 16), jnp.float32),
        pltpu.VMEM((CHUNK, 16), jnp.float32),
    ]
    scratch += [pltpu.SemaphoreType.DMA] * (2 * NBUF)
    scratch.append(pltpu.VMEM_SHARED((np_rows, dim), jnp.float32))
    if with_count:
      scratch.append(pltpu.VMEM_SHARED((np_rows, 16), jnp.float32))
    fn = pl.kernel(
        body,
        out_type=tuple(out_type),
        mesh=plsc.VectorSubcoreMesh(core_axis_name="c", subcore_axis_name="s",
                                    num_cores=NC, num_subcores=NS),
        scratch_types=tuple(scratch),
        compiler_params=pltpu.CompilerParams(use_tc_tiling_on_sc=False),
    )
    return fn(h, src2d, dst2d)

  return call


def _tc_project(x, wl, wr, b):
  rows, d = x.shape
  h = wl.shape[1]
  hh = h // 2

  def body(x_ref, wl_ref, wr_ref, b_ref, ha_ref, hb_ref, r_ref):
    xb = x_ref[...]
    hv = jnp.dot(xb, wl_ref[...], preferred_element_type=jnp.float32)
    ha_ref[...] = hv[:, :hh]
    hb_ref[...] = hv[:, hh:]
    r_ref[...] = (jnp.dot(xb, wr_ref[...], preferred_element_type=jnp.float32)
                  + b_ref[...])

  return pl.pallas_call(
      body,
      grid=(rows // RB,),
      in_specs=[
          pl.BlockSpec((RB, d), lambda i: (i, 0)),
          pl.BlockSpec((d, h), lambda i: (0, 0)),
          pl.BlockSpec((d, h), lambda i: (0, 0)),
          pl.BlockSpec((1, h), lambda i: (0, 0)),
      ],
      out_specs=[
          pl.BlockSpec((RB, hh), lambda i: (i, 0)),
          pl.BlockSpec((RB, hh), lambda i: (i, 0)),
          pl.BlockSpec((RB, h), lambda i: (i, 0)),
      ],
      out_shape=[
          jax.ShapeDtypeStruct((rows, hh), jnp.float32),
          jax.ShapeDtypeStruct((rows, hh), jnp.float32),
          jax.ShapeDtypeStruct((rows, h), jnp.float32),
      ],
  )(x, wl, wr, b)


def _tc_mid(agga, aggb, cnt, r1, wl, wr, b):
  rows, d = r1.shape
  dh = d // 2
  h = wl.shape[1]

  def body(agga_ref, aggb_ref, cnt_ref, r1_ref, wl_ref, wr_ref, b_ref,
           h_ref, r_ref):
    cb = cnt_ref[0] + cnt_ref[1]
    inv = 1.0 / jnp.maximum(cb[:, :1], 1.0)
    mean = jnp.concatenate(
        [agga_ref[0] + agga_ref[1], aggb_ref[0] + aggb_ref[1]], axis=1) * inv
    z = jnp.maximum(mean + r1_ref[...], 0.0)
    h_ref[...] = jnp.dot(z, wl_ref[...], preferred_element_type=jnp.float32)
    r_ref[...] = (jnp.dot(z, wr_ref[...], preferred_element_type=jnp.float32)
                  + b_ref[...])

  return pl.pallas_call(
      body,
      grid=(rows // RB,),
      in_specs=[
          pl.BlockSpec((NC, RB, dh), lambda i: (0, i, 0)),
          pl.BlockSpec((NC, RB, dh), lambda i: (0, i, 0)),
          pl.BlockSpec((NC, RB, 16), lambda i: (0, i, 0)),
          pl.BlockSpec((RB, d), lambda i: (i, 0)),
          pl.BlockSpec((d, h), lambda i: (0, 0)),
          pl.BlockSpec((d, h), lambda i: (0, 0)),
          pl.BlockSpec((1, h), lambda i: (0, 0)),
      ],
      out_specs=[
          pl.BlockSpec((RB, h), lambda i: (i, 0)),
          pl.BlockSpec((RB, h), lambda i: (i, 0)),
      ],
      out_shape=[
          jax.ShapeDtypeStruct((rows, h), jnp.float32),
          jax.ShapeDtypeStruct((rows, h), jnp.float32),
      ],
  )(agga, aggb, cnt, r1, wl, wr, b)


def _tc_final(agg, cnt, r2, n):
  h = r2.shape[1]
  rb = n // 5

  def body(agg_ref, cnt_ref, r2_ref, o_ref):
    cb = cnt_ref[0] + cnt_ref[1]
    inv = 1.0 / jnp.maximum(cb[:, :1], 1.0)
    o_ref[...] = (agg_ref[0] + agg_ref[1]) * inv + r2_ref[...]

  return pl.pallas_call(
      body,
      grid=(5,),
      in_specs=[
          pl.BlockSpec((NC, rb, h), lambda i: (0, i, 0)),
          pl.BlockSpec((NC, rb, 16), lambda i: (0, i, 0)),
          pl.BlockSpec((rb, h), lambda i: (i, 0)),
      ],
      out_specs=pl.BlockSpec((rb, h), lambda i: (i, 0)),
      out_shape=jax.ShapeDtypeStruct((n, h), jnp.float32),
  )(agg, cnt, r2)


@jax.jit
def kernel(x, edge_index, W1_l, b1, W1_r, W2_l, b2, W2_r):
  n, d = x.shape
  e = edge_index.shape[1]

  np_rows = ((n + 240 + NS * CHUNK - 1) // (NS * CHUNK)) * (NS * CHUNK)
  ep = ((e + NW * CW * CHUNK - 1) // (NW * CW * CHUNK)) * (NW * CW * CHUNK)
  pad_e = ep - e
  n_dummy = np_rows - n

  src = edge_index[0]
  dst = edge_index[1]
  pad_ids = np.arange(pad_e, dtype=np.int32)
  src_p = jnp.concatenate([src, jnp.asarray(pad_ids % n)])
  dst_p = jnp.concatenate([dst, jnp.asarray(n + pad_ids % n_dummy)])
  src2d = src_p.reshape(NW * CW * (ep // (NW * CW * CHUNK)), CHUNK)
  dst2d = dst_p.reshape(src2d.shape)

  x_p = jnp.pad(x, ((0, np_rows - n), (0, 0)))
  b1r = b1.reshape(1, -1)
  b2r = b2.reshape(1, -1)

  h1a, h1b, r1 = _tc_project(x_p, W1_l.T, W1_r.T, b1r)
  seg = _sc_segment_sum(h1a.shape[1], True)
  seg_nc = _sc_segment_sum(h1a.shape[1], False)
  agg1a, cnt = seg(h1a, src2d, dst2d)
  (agg1b,) = seg_nc(h1b, src2d, dst2d)
  h2, r2 = _tc_mid(agg1a, agg1b, cnt, r1, W2_l.T, W2_r.T, b2r)
  (agg2,) = seg_nc(h2, src2d, dst2d)
  return _tc_final(agg2, cnt, r2, n)


# speedup vs baseline: 13.5882x; 1.0286x over previous
import functools

import jax
import jax.numpy as jnp
import numpy as np
from jax import lax
from jax.experimental import pallas as pl
from jax.experimental.pallas import tpu as pltpu
from jax.experimental.pallas import tpu_sc as plsc

NC = 2
NS = 16
NW = NC * NS
CHUNK = 128
CW = 80
NBUF = 4
AHEAD = 2
RB = 1280


def _sc_segment_sum(dim, with_count):
  def body(h_hbm, src_hbm, dst_hbm, *rest):
    if with_count:
      agg_out, cnt_out = rest[0], rest[1]
      rest = rest[2:]
    else:
      agg_out = rest[0]
      cnt_out = None
      rest = rest[1:]
    idx_src, idx_dst, rows_v, ones_v, zcnt_v = rest[:5]
    gsem = rest[5:5 + NBUF]
    ssem = rest[5 + NBUF:5 + 2 * NBUF]
    csem = rest[5 + 2 * NBUF]
    acc_sh = rest[6 + 2 * NBUF]
    cnt_sh = rest[7 + 2 * NBUF] if with_count else None

    np_rows = h_hbm.shape[0]
    rows_per_tile = np_rows // NS

    c = lax.axis_index("c")
    s = lax.axis_index("s")
    wid = s * NC + c

    def gather_start(i, b):
      pltpu.async_copy(h_hbm.at[idx_src.at[i]], rows_v.at[b], gsem[b])

    def gather_wait(i, b):
      pltpu.make_async_copy(h_hbm.at[idx_src.at[i]], rows_v.at[b],
                            gsem[b]).wait()

    def scatter_start(i, b):
      pltpu.async_copy(rows_v.at[b], acc_sh.at[idx_dst.at[i]], ssem[b],
                       add=True)

    def scatter_wait(i, b):
      pltpu.make_async_copy(rows_v.at[b], acc_sh.at[idx_dst.at[i]],
                            ssem[b]).wait()

    src_cp = pltpu.make_async_copy(src_hbm.at[pl.ds(wid * CW, CW), :],
                                   idx_src, gsem[2])
    dst_cp = pltpu.make_async_copy(dst_hbm.at[pl.ds(wid * CW, CW), :],
                                   idx_dst, gsem[3])
    pltpu.async_copy(src_hbm.at[pl.ds(wid * CW, CW), :], idx_src, gsem[2])
    pltpu.async_copy(dst_hbm.at[pl.ds(wid * CW, CW), :], idx_dst, gsem[3])

    zf = jnp.zeros((16,), jnp.float32)
    of = jnp.ones((16,), jnp.float32)

    def init_row(r, carry):
      for j in range(dim // 16):
        rows_v[NBUF - 1, r, pl.ds(j * 16, 16)] = zf
      ones_v[r, pl.ds(0, 16)] = of
      zcnt_v[r, pl.ds(0, 16)] = zf
      return carry

    lax.fori_loop(0, CHUNK, init_row, 0)

    src_cp.wait()
    for i in range(AHEAD):
      gather_start(i, i % NBUF)

    tbase = s * rows_per_tile
    for k in range(rows_per_tile // CHUNK):
      pltpu.async_copy(rows_v.at[NBUF - 1],
                       acc_sh.at[pl.ds(tbase + k * CHUNK, CHUNK), :], ssem[0])
      if with_count:
        pltpu.async_copy(zcnt_v,
                         cnt_sh.at[pl.ds(tbase + k * CHUNK, CHUNK), :],
                         ssem[1])
    dst_cp.wait()
    for k in range(rows_per_tile // CHUNK):
      pltpu.make_async_copy(
          rows_v.at[NBUF - 1],
          acc_sh.at[pl.ds(tbase + k * CHUNK, CHUNK), :], ssem[0]).wait()
      if with_count:
        pltpu.make_async_copy(
            zcnt_v, cnt_sh.at[pl.ds(tbase + k * CHUNK, CHUNK), :],
            ssem[1]).wait()
    plsc.subcore_barrier()

    def step(i, b, first, last):
      j = i + AHEAD
      bj = (b + AHEAD) % NBUF
      if not first and not last:
        scatter_wait(j - NBUF, bj)
      if not last:
        gather_start(j, bj)
      gather_wait(i, b)
      scatter_start(i, b)
      if with_count:
        pltpu.async_copy(ones_v, cnt_sh.at[idx_dst.at[i]], csem, add=True)

    for b in range(NBUF):
      step(b, b, first=(b < NBUF - AHEAD), last=False)

    def group(g, carry):
      for b in range(NBUF):
        step(g * NBUF + b, b, first=False, last=False)
      return carry

    lax.fori_loop(1, CW // NBUF - 1, group, 0)

    g = CW // NBUF - 1
    for b in range(NBUF):
      i = g * NBUF + b
      step(i, b, first=False, last=(i + AHEAD >= CW))

    for i in range(CW - NBUF, CW):
      scatter_wait(i, i % NBUF)

    if with_count:
      def drain_cnt(i, carry):
        pltpu.make_async_copy(ones_v, cnt_sh.at[idx_dst.at[0]], csem).wait()
        return carry
      lax.fori_loop(0, CW, drain_cnt, 0)

    plsc.subcore_barrier()

    pltpu.sync_copy(acc_sh.at[pl.ds(tbase, rows_per_tile), :],
                    agg_out.at[c, pl.ds(tbase, rows_per_tile), :])
    if with_count:
      pltpu.sync_copy(cnt_sh.at[pl.ds(tbase, rows_per_tile), :],
                      cnt_out.at[c, pl.ds(tbase, rows_per_tile), :])

  def call(h, src2d, dst2d):
    np_rows = h.shape[0]
    out_type = [jax.ShapeDtypeStruct((NC, np_rows, dim), jnp.float32)]
    if with_count:
      out_type.append(jax.ShapeDtypeStruct((NC, np_rows, 16), jnp.float32))
    scratch = [
        pltpu.VMEM((CW, CHUNK), jnp.int32),
        pltpu.VMEM((CW, CHUNK), jnp.int32),
        pltpu.VMEM((NBUF, CHUNK, dim), jnp.float32),
        pltpu.VMEM((CHUNK, 16), jnp.float32),
        pltpu.VMEM((CHUNK, 16), jnp.float32),
    ]
    scratch += [pltpu.SemaphoreType.DMA] * (2 * NBUF + 1)
    scratch.append(pltpu.VMEM_SHARED((np_rows, dim), jnp.float32))
    if with_count:
      scratch.append(pltpu.VMEM_SHARED((np_rows, 16), jnp.float32))
    fn = pl.kernel(
        body,
        out_type=tuple(out_type),
        mesh=plsc.VectorSubcoreMesh(core_axis_name="c", subcore_axis_name="s",
                                    num_cores=NC, num_subcores=NS),
        scratch_types=tuple(scratch),
        compiler_params=pltpu.CompilerParams(use_tc_tiling_on_sc=False),
    )
    return fn(h, src2d, dst2d)

  return call


def _tc_project(x, wl, wr, b):
  rows, d = x.shape
  h = wl.shape[1]
  hh = h // 2

  def body(x_ref, wl_ref, wr_ref, b_ref, ha_ref, hb_ref, r_ref):
    xb = x_ref[...]
    hv = jnp.dot(xb, wl_ref[...], preferred_element_type=jnp.float32)
    ha_ref[...] = hv[:, :hh]
    hb_ref[...] = hv[:, hh:]
    r_ref[...] = (jnp.dot(xb, wr_ref[...], preferred_element_type=jnp.float32)
                  + b_ref[...])

  return pl.pallas_call(
      body,
      grid=(rows // RB,),
      in_specs=[
          pl.BlockSpec((RB, d), lambda i: (i, 0)),
          pl.BlockSpec((d, h), lambda i: (0, 0)),
          pl.BlockSpec((d, h), lambda i: (0, 0)),
          pl.BlockSpec((1, h), lambda i: (0, 0)),
      ],
      out_specs=[
          pl.BlockSpec((RB, hh), lambda i: (i, 0)),
          pl.BlockSpec((RB, hh), lambda i: (i, 0)),
          pl.BlockSpec((RB, h), lambda i: (i, 0)),
      ],
      out_shape=[
          jax.ShapeDtypeStruct((rows, hh), jnp.float32),
          jax.ShapeDtypeStruct((rows, hh), jnp.float32),
          jax.ShapeDtypeStruct((rows, h), jnp.float32),
      ],
  )(x, wl, wr, b)


def _tc_mid(agga, aggb, cnt, r1, wl, wr, b):
  rows, d = r1.shape
  dh = d // 2
  h = wl.shape[1]

  def body(agga_ref, aggb_ref, cnt_ref, r1_ref, wl_ref, wr_ref, b_ref,
           h_ref, r_ref):
    cb = cnt_ref[0] + cnt_ref[1]
    inv = 1.0 / jnp.maximum(cb[:, :1], 1.0)
    mean = jnp.concatenate(
        [agga_ref[0] + agga_ref[1], aggb_ref[0] + aggb_ref[1]], axis=1) * inv
    z = jnp.maximum(mean + r1_ref[...], 0.0)
    h_ref[...] = jnp.dot(z, wl_ref[...], preferred_element_type=jnp.float32)
    r_ref[...] = (jnp.dot(z, wr_ref[...], preferred_element_type=jnp.float32)
                  + b_ref[...])

  return pl.pallas_call(
      body,
      grid=(rows // RB,),
      in_specs=[
          pl.BlockSpec((NC, RB, dh), lambda i: (0, i, 0)),
          pl.BlockSpec((NC, RB, dh), lambda i: (0, i, 0)),
          pl.BlockSpec((NC, RB, 16), lambda i: (0, i, 0)),
          pl.BlockSpec((RB, d), lambda i: (i, 0)),
          pl.BlockSpec((d, h), lambda i: (0, 0)),
          pl.BlockSpec((d, h), lambda i: (0, 0)),
          pl.BlockSpec((1, h), lambda i: (0, 0)),
      ],
      out_specs=[
          pl.BlockSpec((RB, h), lambda i: (i, 0)),
          pl.BlockSpec((RB, h), lambda i: (i, 0)),
      ],
      out_shape=[
          jax.ShapeDtypeStruct((rows, h), jnp.float32),
          jax.ShapeDtypeStruct((rows, h), jnp.float32),
      ],
  )(agga, aggb, cnt, r1, wl, wr, b)


def _tc_final(agg, cnt, r2, n):
  h = r2.shape[1]
  rb = n // 5

  def body(agg_ref, cnt_ref, r2_ref, o_ref):
    cb = cnt_ref[0] + cnt_ref[1]
    inv = 1.0 / jnp.maximum(cb[:, :1], 1.0)
    o_ref[...] = (agg_ref[0] + agg_ref[1]) * inv + r2_ref[...]

  return pl.pallas_call(
      body,
      grid=(5,),
      in_specs=[
          pl.BlockSpec((NC, rb, h), lambda i: (0, i, 0)),
          pl.BlockSpec((NC, rb, 16), lambda i: (0, i, 0)),
          pl.BlockSpec((rb, h), lambda i: (i, 0)),
      ],
      out_specs=pl.BlockSpec((rb, h), lambda i: (i, 0)),
      out_shape=jax.ShapeDtypeStruct((n, h), jnp.float32),
  )(agg, cnt, r2)


@jax.jit
def kernel(x, edge_index, W1_l, b1, W1_r, W2_l, b2, W2_r):
  n, d = x.shape
  e = edge_index.shape[1]

  np_rows = ((n + 240 + NS * CHUNK - 1) // (NS * CHUNK)) * (NS * CHUNK)
  ep = ((e + NW * CW * CHUNK - 1) // (NW * CW * CHUNK)) * (NW * CW * CHUNK)
  pad_e = ep - e
  n_dummy = np_rows - n

  src = edge_index[0]
  dst = edge_index[1]
  pad_ids = np.arange(pad_e, dtype=np.int32)
  src_p = jnp.concatenate([src, jnp.asarray(pad_ids % n)])
  dst_p = jnp.concatenate([dst, jnp.asarray(n + pad_ids % n_dummy)])
  src2d = src_p.reshape(NW * CW * (ep // (NW * CW * CHUNK)), CHUNK)
  dst2d = dst_p.reshape(src2d.shape)

  x_p = jnp.pad(x, ((0, np_rows - n), (0, 0)))
  b1r = b1.reshape(1, -1)
  b2r = b2.reshape(1, -1)

  h1a, h1b, r1 = _tc_project(x_p, W1_l.T, W1_r.T, b1r)
  seg = _sc_segment_sum(d // 2, True)
  seg_nc = _sc_segment_sum(d // 2, False)
  agg1a, cnt = seg(h1a, src2d, dst2d)
  (agg1b,) = seg_nc(h1b, src2d, dst2d)
  h2, r2 = _tc_mid(agg1a, agg1b, cnt, r1, W2_l.T, W2_r.T, b2r)
  (agg2,) = seg_nc(h2, src2d, dst2d)
  return _tc_final(agg2, cnt, r2, n)


# speedup vs baseline: 14.4808x; 1.0657x over previous
import functools

import jax
import jax.numpy as jnp
import numpy as np
from jax import lax
from jax.experimental import pallas as pl
from jax.experimental.pallas import tpu as pltpu
from jax.experimental.pallas import tpu_sc as plsc

NC = 2
NS = 16
NW = NC * NS
CHUNK = 128
CW = 80
NBUF = 4
AHEAD = 2
RB = 1280


def _sc_segment_sum(dim, with_count):
  def body(h_hbm, src_hbm, dst_hbm, *rest):
    if with_count:
      agg_out, cnt_out = rest[0], rest[1]
      rest = rest[2:]
    else:
      agg_out = rest[0]
      cnt_out = None
      rest = rest[1:]
    idx_src, idx_dst, rows_v, ones_v, zcnt_v = rest[:5]
    gsem = rest[5:5 + NBUF]
    ssem = rest[5 + NBUF:5 + 2 * NBUF]
    csem = rest[5 + 2 * NBUF]
    acc_sh = rest[6 + 2 * NBUF]
    cnt_sh = rest[7 + 2 * NBUF] if with_count else None

    np_rows = h_hbm.shape[0]
    rows_per_tile = np_rows // NS

    c = lax.axis_index("c")
    s = lax.axis_index("s")
    wid = s * NC + c

    def gather_start(i, b):
      pltpu.async_copy(h_hbm.at[idx_src.at[i]], rows_v.at[b], gsem[b])

    def gather_wait(i, b):
      pltpu.make_async_copy(h_hbm.at[idx_src.at[i]], rows_v.at[b],
                            gsem[b]).wait()

    def scatter_start(i, b):
      pltpu.async_copy(rows_v.at[b], acc_sh.at[idx_dst.at[i]], ssem[b],
                       add=True)

    def scatter_wait(i, b):
      pltpu.make_async_copy(rows_v.at[b], acc_sh.at[idx_dst.at[i]],
                            ssem[b]).wait()

    src_cp = pltpu.make_async_copy(src_hbm.at[pl.ds(wid * CW, CW), :],
                                   idx_src, gsem[2])
    dst_cp = pltpu.make_async_copy(dst_hbm.at[pl.ds(wid * CW, CW), :],
                                   idx_dst, gsem[3])
    pltpu.async_copy(src_hbm.at[pl.ds(wid * CW, CW), :], idx_src, gsem[2])
    pltpu.async_copy(dst_hbm.at[pl.ds(wid * CW, CW), :], idx_dst, gsem[3])

    zf = jnp.zeros((16,), jnp.float32)
    of = jnp.ones((16,), jnp.float32)

    def init_row(r, carry):
      for j in range(dim // 16):
        rows_v[NBUF - 1, r, pl.ds(j * 16, 16)] = zf
      ones_v[r, pl.ds(0, 16)] = of
      zcnt_v[r, pl.ds(0, 16)] = zf
      return carry

    lax.fori_loop(0, CHUNK, init_row, 0)

    src_cp.wait()
    for i in range(AHEAD):
      gather_start(i, i % NBUF)

    tbase = s * rows_per_tile
    for k in range(rows_per_tile // CHUNK):
      pltpu.async_copy(rows_v.at[NBUF - 1],
                       acc_sh.at[pl.ds(tbase + k * CHUNK, CHUNK), :], ssem[0])
      if with_count:
        pltpu.async_copy(zcnt_v,
                         cnt_sh.at[pl.ds(tbase + k * CHUNK, CHUNK), :],
                         ssem[1])
    dst_cp.wait()
    for k in range(rows_per_tile // CHUNK):
      pltpu.make_async_copy(
          rows_v.at[NBUF - 1],
          acc_sh.at[pl.ds(tbase + k * CHUNK, CHUNK), :], ssem[0]).wait()
      if with_count:
        pltpu.make_async_copy(
            zcnt_v, cnt_sh.at[pl.ds(tbase + k * CHUNK, CHUNK), :],
            ssem[1]).wait()
    plsc.subcore_barrier()

    def step(i, b, first, last):
      j = i + AHEAD
      bj = (b + AHEAD) % NBUF
      if not first and not last:
        scatter_wait(j - NBUF, bj)
      if not last:
        gather_start(j, bj)
      gather_wait(i, b)
      scatter_start(i, b)
      if with_count:
        pltpu.async_copy(ones_v, cnt_sh.at[idx_dst.at[i]], csem, add=True)

    for b in range(NBUF):
      step(b, b, first=(b < NBUF - AHEAD), last=False)

    def group(g, carry):
      for b in range(NBUF):
        step(g * NBUF + b, b, first=False, last=False)
      return carry

    lax.fori_loop(1, CW // NBUF - 1, group, 0)

    g = CW // NBUF - 1
    for b in range(NBUF):
      i = g * NBUF + b
      step(i, b, first=False, last=(i + AHEAD >= CW))

    for i in range(CW - NBUF, CW):
      scatter_wait(i, i % NBUF)

    if with_count:
      def drain_cnt(i, carry):
        pltpu.make_async_copy(ones_v, cnt_sh.at[idx_dst.at[0]], csem).wait()
        return carry
      lax.fori_loop(0, CW, drain_cnt, 0)

    plsc.subcore_barrier()

    pltpu.sync_copy(acc_sh.at[pl.ds(tbase, rows_per_tile), :],
                    agg_out.at[c, pl.ds(tbase, rows_per_tile), :])
    if with_count:
      pltpu.sync_copy(cnt_sh.at[pl.ds(tbase, rows_per_tile), :],
                      cnt_out.at[c, pl.ds(tbase, rows_per_tile), :])

  def call(h, src2d, dst2d):
    np_rows = h.shape[0]
    out_type = [jax.ShapeDtypeStruct((NC, np_rows, dim), jnp.float32)]
    if with_count:
      out_type.append(jax.ShapeDtypeStruct((NC, np_rows, 16), jnp.float32))
    scratch = [
        pltpu.VMEM((CW, CHUNK), jnp.int32),
        pltpu.VMEM((CW, CHUNK), jnp.int32),
        pltpu.VMEM((NBUF, CHUNK, dim), jnp.float32),
        pltpu.VMEM((CHUNK, 16), jnp.float32),
        pltpu.VMEM((CHUNK, 16), jnp.float32),
    ]
    scratch += [pltpu.SemaphoreType.DMA] * (2 * NBUF + 1)
    scratch.append(pltpu.VMEM_SHARED((np_rows, dim), jnp.float32))
    if with_count:
      scratch.append(pltpu.VMEM_SHARED((np_rows, 16), jnp.float32))
    fn = pl.kernel(
        body,
        out_type=tuple(out_type),
        mesh=plsc.VectorSubcoreMesh(core_axis_name="c", subcore_axis_name="s",
                                    num_cores=NC, num_subcores=NS),
        scratch_types=tuple(scratch),
        compiler_params=pltpu.CompilerParams(use_tc_tiling_on_sc=False),
    )
    return fn(h, src2d, dst2d)

  return call


def _tc_project(x, wl, wr, b):
  rows, d = x.shape
  h = wl.shape[1]
  hh = h // 2

  def body(x_ref, wl_ref, wr_ref, b_ref, ha_ref, hb_ref, r_ref):
    xb = x_ref[...]
    hv = jnp.dot(xb, wl_ref[...], preferred_element_type=jnp.float32)
    ha_ref[...] = hv[:, :hh]
    hb_ref[...] = hv[:, hh:]
    r_ref[...] = (jnp.dot(xb, wr_ref[...], preferred_element_type=jnp.float32)
                  + b_ref[...])

  return pl.pallas_call(
      body,
      grid=(rows // RB,),
      in_specs=[
          pl.BlockSpec((RB, d), lambda i: (i, 0)),
          pl.BlockSpec((d, h), lambda i: (0, 0)),
          pl.BlockSpec((d, h), lambda i: (0, 0)),
          pl.BlockSpec((1, h), lambda i: (0, 0)),
      ],
      out_specs=[
          pl.BlockSpec((RB, hh), lambda i: (i, 0)),
          pl.BlockSpec((RB, hh), lambda i: (i, 0)),
          pl.BlockSpec((RB, h), lambda i: (i, 0)),
      ],
      out_shape=[
          jax.ShapeDtypeStruct((rows, hh), jnp.float32),
          jax.ShapeDtypeStruct((rows, hh), jnp.float32),
          jax.ShapeDtypeStruct((rows, h), jnp.float32),
      ],
  )(x, wl, wr, b)


def _unpack2(v, w):
  s = jnp.concatenate([v[:, :w][:, None, :], v[:, w:][:, None, :]], axis=1)
  return s.reshape(2 * v.shape[0], w)


def _tc_mid(agga, aggb, cnt, r1, wl, wr, b):
  rows, d = r1.shape
  dh = d // 2
  h = wl.shape[1]

  def body(agga_ref, aggb_ref, cnt_ref, r1_ref, wl_ref, wr_ref, b_ref,
           h_ref, r_ref):
    cb = cnt_ref[0] + cnt_ref[1]
    inv = 1.0 / jnp.maximum(cb[:, :1], 1.0)
    mean = jnp.concatenate(
        [_unpack2(agga_ref[0] + agga_ref[1], dh),
         _unpack2(aggb_ref[0] + aggb_ref[1], dh)], axis=1) * inv
    z = jnp.maximum(mean + r1_ref[...], 0.0)
    h_ref[...] = jnp.dot(z, wl_ref[...], preferred_element_type=jnp.float32)
    r_ref[...] = (jnp.dot(z, wr_ref[...], preferred_element_type=jnp.float32)
                  + b_ref[...])

  return pl.pallas_call(
      body,
      grid=(rows // RB,),
      in_specs=[
          pl.BlockSpec((NC, RB // 2, 2 * dh), lambda i: (0, i, 0)),
          pl.BlockSpec((NC, RB // 2, 2 * dh), lambda i: (0, i, 0)),
          pl.BlockSpec((NC, RB, 16), lambda i: (0, i, 0)),
          pl.BlockSpec((RB, d), lambda i: (i, 0)),
          pl.BlockSpec((d, h), lambda i: (0, 0)),
          pl.BlockSpec((d, h), lambda i: (0, 0)),
          pl.BlockSpec((1, h), lambda i: (0, 0)),
      ],
      out_specs=[
          pl.BlockSpec((RB, h), lambda i: (i, 0)),
          pl.BlockSpec((RB, h), lambda i: (i, 0)),
      ],
      out_shape=[
          jax.ShapeDtypeStruct((rows, h), jnp.float32),
          jax.ShapeDtypeStruct((rows, h), jnp.float32),
      ],
  )(agga, aggb, cnt, r1, wl, wr, b)


def _tc_final(agg, cnt, r2, n):
  h = r2.shape[1]
  rb = n // 5

  def body(agg_ref, cnt_ref, r2_ref, o_ref):
    cb = cnt_ref[0] + cnt_ref[1]
    inv = 1.0 / jnp.maximum(cb[:, :1], 1.0)
    o_ref[...] = _unpack2(agg_ref[0] + agg_ref[1], h) * inv + r2_ref[...]

  return pl.pallas_call(
      body,
      grid=(5,),
      in_specs=[
          pl.BlockSpec((NC, rb // 2, 2 * h), lambda i: (0, i, 0)),
          pl.BlockSpec((NC, rb, 16), lambda i: (0, i, 0)),
          pl.BlockSpec((rb, h), lambda i: (i, 0)),
      ],
      out_specs=pl.BlockSpec((rb, h), lambda i: (i, 0)),
      out_shape=jax.ShapeDtypeStruct((n, h), jnp.float32),
  )(agg, cnt, r2)


@jax.jit
def kernel(x, edge_index, W1_l, b1, W1_r, W2_l, b2, W2_r):
  n, d = x.shape
  e = edge_index.shape[1]

  np_rows = ((n + 240 + NS * CHUNK - 1) // (NS * CHUNK)) * (NS * CHUNK)
  ep = ((e + NW * CW * CHUNK - 1) // (NW * CW * CHUNK)) * (NW * CW * CHUNK)
  pad_e = ep - e
  n_dummy = np_rows - n

  src = edge_index[0]
  dst = edge_index[1]
  pad_ids = np.arange(pad_e, dtype=np.int32)
  src_p = jnp.concatenate([src, jnp.asarray(pad_ids % n)])
  dst_p = jnp.concatenate([dst, jnp.asarray(n + pad_ids % n_dummy)])
  src2d = src_p.reshape(NW * CW * (ep // (NW * CW * CHUNK)), CHUNK)
  dst2d = dst_p.reshape(src2d.shape)

  x_p = jnp.pad(x, ((0, np_rows - n), (0, 0)))
  b1r = b1.reshape(1, -1)
  b2r = b2.reshape(1, -1)

  h1a, h1b, r1 = _tc_project(x_p, W1_l.T, W1_r.T, b1r)
  seg = _sc_segment_sum(d // 2, True)
  seg_nc = _sc_segment_sum(d // 2, False)
  (agg1b,) = seg_nc(h1b, src2d, dst2d)
  agg1a, cnt = seg(h1a, src2d, dst2d)
  h2, r2 = _tc_mid(agg1a.reshape(NC, np_rows // 2, 128),
                   agg1b.reshape(NC, np_rows // 2, 128),
                   cnt, r1, W2_l.T, W2_r.T, b2r)
  (agg2,) = seg_nc(h2, src2d, dst2d)
  return _tc_final(agg2.reshape(NC, np_rows // 2, 128), cnt, r2, n)


# speedup vs baseline: 15.0159x; 1.0370x over previous
import functools

import jax
import jax.numpy as jnp
import numpy as np
from jax import lax
from jax.experimental import pallas as pl
from jax.experimental.pallas import tpu as pltpu
from jax.experimental.pallas import tpu_sc as plsc

NC = 2
NS = 16
NW = NC * NS
CHUNK = 128
CW = 80
NBUF = 4
AHEAD = 2
RB = 1280


def _sc_segment_sum(dim, with_count):
  def body(h_hbm, src_hbm, dst_hbm, *rest):
    if with_count:
      agg_out, cnt_out = rest[0], rest[1]
      rest = rest[2:]
    else:
      agg_out = rest[0]
      cnt_out = None
      rest = rest[1:]
    idx_src, idx_dst, rows_v, ones_v, zcnt_v = rest[:5]
    gsem = rest[5:5 + NBUF]
    ssem = rest[5 + NBUF:5 + 2 * NBUF]
    csem = rest[5 + 2 * NBUF]
    acc_sh = rest[6 + 2 * NBUF]
    cnt_sh = rest[7 + 2 * NBUF] if with_count else None

    np_rows = h_hbm.shape[0]
    rows_per_tile = np_rows // NS

    c = lax.axis_index("c")
    s = lax.axis_index("s")
    wid = s * NC + c

    def gather_start(i, b):
      pltpu.async_copy(h_hbm.at[idx_src.at[i]], rows_v.at[b], gsem[b])

    def gather_wait(i, b):
      pltpu.make_async_copy(h_hbm.at[idx_src.at[i]], rows_v.at[b],
                            gsem[b]).wait()

    def scatter_start(i, b):
      pltpu.async_copy(rows_v.at[b], acc_sh.at[idx_dst.at[i]], ssem[b],
                       add=True)

    def scatter_wait(i, b):
      pltpu.make_async_copy(rows_v.at[b], acc_sh.at[idx_dst.at[i]],
                            ssem[b]).wait()

    src_cp = pltpu.make_async_copy(src_hbm.at[pl.ds(wid * CW, CW), :],
                                   idx_src, gsem[2])
    dst_cp = pltpu.make_async_copy(dst_hbm.at[pl.ds(wid * CW, CW), :],
                                   idx_dst, gsem[3])
    pltpu.async_copy(src_hbm.at[pl.ds(wid * CW, CW), :], idx_src, gsem[2])
    pltpu.async_copy(dst_hbm.at[pl.ds(wid * CW, CW), :], idx_dst, gsem[3])

    zf = jnp.zeros((16,), jnp.float32)
    of = jnp.ones((16,), jnp.float32)

    def init_row(r, carry):
      for j in range(dim // 16):
        rows_v[NBUF - 1, r, pl.ds(j * 16, 16)] = zf
      ones_v[r, pl.ds(0, 16)] = of
      zcnt_v[r, pl.ds(0, 16)] = zf
      return carry

    lax.fori_loop(0, CHUNK, init_row, 0)

    src_cp.wait()
    for i in range(AHEAD):
      gather_start(i, i % NBUF)

    tbase = s * rows_per_tile
    for k in range(rows_per_tile // CHUNK):
      pltpu.async_copy(rows_v.at[NBUF - 1],
                       acc_sh.at[pl.ds(tbase + k * CHUNK, CHUNK), :], ssem[0])
      if with_count:
        pltpu.async_copy(zcnt_v,
                         cnt_sh.at[pl.ds(tbase + k * CHUNK, CHUNK), :],
                         ssem[1])
    dst_cp.wait()
    for k in range(rows_per_tile // CHUNK):
      pltpu.make_async_copy(
          rows_v.at[NBUF - 1],
          acc_sh.at[pl.ds(tbase + k * CHUNK, CHUNK), :], ssem[0]).wait()
      if with_count:
        pltpu.make_async_copy(
            zcnt_v, cnt_sh.at[pl.ds(tbase + k * CHUNK, CHUNK), :],
            ssem[1]).wait()
    plsc.subcore_barrier()

    def step(i, b, first, last):
      j = i + AHEAD
      bj = (b + AHEAD) % NBUF
      if not first and not last:
        scatter_wait(j - NBUF, bj)
      if not last:
        gather_start(j, bj)
      gather_wait(i, b)
      scatter_start(i, b)
      if with_count:
        pltpu.async_copy(ones_v, cnt_sh.at[idx_dst.at[i]], csem, add=True)

    for b in range(NBUF):
      step(b, b, first=(b < NBUF - AHEAD), last=False)

    def group(g, carry):
      for b in range(NBUF):
        step(g * NBUF + b, b, first=False, last=False)
      return carry

    lax.fori_loop(1, CW // NBUF - 1, group, 0)

    g = CW // NBUF - 1
    for b in range(NBUF):
      i = g * NBUF + b
      step(i, b, first=False, last=(i + AHEAD >= CW))

    for i in range(CW - NBUF, CW):
      scatter_wait(i, i % NBUF)

    if with_count:
      def drain_cnt(i, carry):
        pltpu.make_async_copy(ones_v, cnt_sh.at[idx_dst.at[0]], csem).wait()
        return carry
      lax.fori_loop(0, CW, drain_cnt, 0)

    plsc.subcore_barrier()

    pltpu.sync_copy(acc_sh.at[pl.ds(tbase, rows_per_tile), :],
                    agg_out.at[c, pl.ds(tbase, rows_per_tile), :])
    if with_count:
      pltpu.sync_copy(cnt_sh.at[pl.ds(tbase, rows_per_tile), :],
                      cnt_out.at[c, pl.ds(tbase, rows_per_tile), :])

  def call(h, src2d, dst2d):
    np_rows = h.shape[0]
    out_type = [jax.ShapeDtypeStruct((NC, np_rows, dim), jnp.float32)]
    if with_count:
      out_type.append(jax.ShapeDtypeStruct((NC, np_rows, 16), jnp.float32))
    scratch = [
        pltpu.VMEM((CW, CHUNK), jnp.int32),
        pltpu.VMEM((CW, CHUNK), jnp.int32),
        pltpu.VMEM((NBUF, CHUNK, dim), jnp.float32),
        pltpu.VMEM((CHUNK, 16), jnp.float32),
        pltpu.VMEM((CHUNK, 16), jnp.float32),
    ]
    scratch += [pltpu.SemaphoreType.DMA] * (2 * NBUF + 1)
    scratch.append(pltpu.VMEM_SHARED((np_rows, dim), jnp.float32))
    if with_count:
      scratch.append(pltpu.VMEM_SHARED((np_rows, 16), jnp.float32))
    fn = pl.kernel(
        body,
        out_type=tuple(out_type),
        mesh=plsc.VectorSubcoreMesh(core_axis_name="c", subcore_axis_name="s",
                                    num_cores=NC, num_subcores=NS),
        scratch_types=tuple(scratch),
        compiler_params=pltpu.CompilerParams(use_tc_tiling_on_sc=False),
    )
    return fn(h, src2d, dst2d)

  return call


def _tc_project(x, wl, wr, b):
  rows, d = x.shape
  h = wl.shape[1]
  hh = h // 2

  def body(x_ref, wl_ref, wr_ref, b_ref, ha_ref, hb_ref, r_ref):
    xb = x_ref[...]
    hv = jnp.dot(xb, wl_ref[...], preferred_element_type=jnp.float32)
    ha_ref[...] = _pack2(hv[:, :hh])
    hb_ref[...] = _pack2(hv[:, hh:])
    r_ref[...] = (jnp.dot(xb, wr_ref[...], preferred_element_type=jnp.float32)
                  + b_ref[...])

  return pl.pallas_call(
      body,
      grid=(rows // RB,),
      in_specs=[
          pl.BlockSpec((RB, d), lambda i: (i, 0)),
          pl.BlockSpec((d, h), lambda i: (0, 0)),
          pl.BlockSpec((d, h), lambda i: (0, 0)),
          pl.BlockSpec((1, h), lambda i: (0, 0)),
      ],
      out_specs=[
          pl.BlockSpec((RB // 2, 2 * hh), lambda i: (i, 0)),
          pl.BlockSpec((RB // 2, 2 * hh), lambda i: (i, 0)),
          pl.BlockSpec((RB, h), lambda i: (i, 0)),
      ],
      out_shape=[
          jax.ShapeDtypeStruct((rows // 2, 2 * hh), jnp.float32),
          jax.ShapeDtypeStruct((rows // 2, 2 * hh), jnp.float32),
          jax.ShapeDtypeStruct((rows, h), jnp.float32),
      ],
  )(x, wl, wr, b)


def _pack2(v):
  m2, w = v.shape
  t = v.reshape(m2 // 2, 2, w)
  return jnp.concatenate([t[:, 0, :], t[:, 1, :]], axis=1)


def _unpack2(v, w):
  s = jnp.concatenate([v[:, :w][:, None, :], v[:, w:][:, None, :]], axis=1)
  return s.reshape(2 * v.shape[0], w)


def _tc_mid(agga, aggb, cnt, r1, wl, wr, b):
  rows, d = r1.shape
  dh = d // 2
  h = wl.shape[1]

  def body(agga_ref, aggb_ref, cnt_ref, r1_ref, wl_ref, wr_ref, b_ref,
           h_ref, r_ref):
    cb = cnt_ref[0] + cnt_ref[1]
    inv = 1.0 / jnp.maximum(cb[:, :1], 1.0)
    mean = jnp.concatenate(
        [_unpack2(agga_ref[0] + agga_ref[1], dh),
         _unpack2(aggb_ref[0] + aggb_ref[1], dh)], axis=1) * inv
    z = jnp.maximum(mean + r1_ref[...], 0.0)
    h_ref[...] = _pack2(
        jnp.dot(z, wl_ref[...], preferred_element_type=jnp.float32))
    r_ref[...] = (jnp.dot(z, wr_ref[...], preferred_element_type=jnp.float32)
                  + b_ref[...])

  return pl.pallas_call(
      body,
      grid=(rows // RB,),
      in_specs=[
          pl.BlockSpec((NC, RB // 2, 2 * dh), lambda i: (0, i, 0)),
          pl.BlockSpec((NC, RB // 2, 2 * dh), lambda i: (0, i, 0)),
          pl.BlockSpec((NC, RB, 16), lambda i: (0, i, 0)),
          pl.BlockSpec((RB, d), lambda i: (i, 0)),
          pl.BlockSpec((d, h), lambda i: (0, 0)),
          pl.BlockSpec((d, h), lambda i: (0, 0)),
          pl.BlockSpec((1, h), lambda i: (0, 0)),
      ],
      out_specs=[
          pl.BlockSpec((RB // 2, 2 * h), lambda i: (i, 0)),
          pl.BlockSpec((RB, h), lambda i: (i, 0)),
      ],
      out_shape=[
          jax.ShapeDtypeStruct((rows // 2, 2 * h), jnp.float32),
          jax.ShapeDtypeStruct((rows, h), jnp.float32),
      ],
  )(agga, aggb, cnt, r1, wl, wr, b)


def _tc_final(agg, cnt, r2, n):
  h = r2.shape[1]
  rb = n // 5

  def body(agg_ref, cnt_ref, r2_ref, o_ref):
    cb = cnt_ref[0] + cnt_ref[1]
    inv = 1.0 / jnp.maximum(cb[:, :1], 1.0)
    o_ref[...] = _unpack2(agg_ref[0] + agg_ref[1], h) * inv + r2_ref[...]

  return pl.pallas_call(
      body,
      grid=(5,),
      in_specs=[
          pl.BlockSpec((NC, rb // 2, 2 * h), lambda i: (0, i, 0)),
          pl.BlockSpec((NC, rb, 16), lambda i: (0, i, 0)),
          pl.BlockSpec((rb, h), lambda i: (i, 0)),
      ],
      out_specs=pl.BlockSpec((rb, h), lambda i: (i, 0)),
      out_shape=jax.ShapeDtypeStruct((n, h), jnp.float32),
  )(agg, cnt, r2)


@jax.jit
def kernel(x, edge_index, W1_l, b1, W1_r, W2_l, b2, W2_r):
  n, d = x.shape
  e = edge_index.shape[1]

  np_rows = ((n + 240 + NS * CHUNK - 1) // (NS * CHUNK)) * (NS * CHUNK)
  ep = ((e + NW * CW * CHUNK - 1) // (NW * CW * CHUNK)) * (NW * CW * CHUNK)
  pad_e = ep - e
  n_dummy = np_rows - n

  src = edge_index[0]
  dst = edge_index[1]
  pad_ids = np.arange(pad_e, dtype=np.int32)
  src_p = jnp.concatenate([src, jnp.asarray(pad_ids % n)])
  dst_p = jnp.concatenate([dst, jnp.asarray(n + pad_ids % n_dummy)])
  src2d = src_p.reshape(NW * CW * (ep // (NW * CW * CHUNK)), CHUNK)
  dst2d = dst_p.reshape(src2d.shape)

  x_p = jnp.pad(x, ((0, np_rows - n), (0, 0)))
  b1r = b1.reshape(1, -1)
  b2r = b2.reshape(1, -1)

  h1a, h1b, r1 = _tc_project(x_p, W1_l.T, W1_r.T, b1r)
  seg = _sc_segment_sum(d // 2, True)
  seg_nc = _sc_segment_sum(d // 2, False)
  (agg1b,) = seg_nc(h1b.reshape(np_rows, d // 2), src2d, dst2d)
  agg1a, cnt = seg(h1a.reshape(np_rows, d // 2), src2d, dst2d)
  h2, r2 = _tc_mid(agg1a.reshape(NC, np_rows // 2, 128),
                   agg1b.reshape(NC, np_rows // 2, 128),
                   cnt, r1, W2_l.T, W2_r.T, b2r)
  (agg2,) = seg_nc(h2.reshape(np_rows, h2.shape[1] // 2), src2d, dst2d)
  return _tc_final(agg2.reshape(NC, np_rows // 2, 128), cnt, r2, n)
